# baseline (speedup 1.0000x reference)
import functools

import jax
import jax.numpy as jnp
from jax import lax
from jax.experimental import pallas as pl
from jax.experimental.pallas import tpu as pltpu
from jax.experimental.pallas import tpu_sc as plsc

_BATCH = 4
_SEQ = 8192
_D = 768
_NW = 32
_ROWS_PER_W = _SEQ // _NW
_R = 32
_N_CHUNKS = _ROWS_PER_W // _R
_L = 16


def _sc_body(x_hbm, t_hbm, out_hbm,
             bufx0, bufx1, buft0, buft1, sx0, sx1, st0, st1, so0, so1):
    bufx = (bufx0, bufx1)
    buft = (buft0, buft1)
    sx = (sx0, sx1)
    st = (st0, st1)
    so = (so0, so1)

    wid = lax.axis_index("s") * 2 + lax.axis_index("c")
    r0 = wid * _ROWS_PER_W

    def x_copy(g, bb, xs):
        row = r0 + g * _R
        return pltpu.make_async_copy(
            x_hbm.at[bb, pl.ds(row, _R), :], bufx[xs], sx[xs])

    def t_copy(g, ts):
        row = r0 + g * _R
        return pltpu.make_async_copy(
            t_hbm.at[pl.ds(row, _R), :], buft[ts], st[ts])

    def out_copy(g, bb, xs):
        row = r0 + g * _R
        return pltpu.make_async_copy(
            bufx[xs], out_hbm.at[bb, pl.ds(row, _R), :], so[xs])

    def accumulate(xs, ts):
        half = _D // 2

        @plsc.parallel_loop(0, 2 * _R, 1, unroll=2)
        def _(r2):
            r = r2 >> 1
            c0 = (r2 & 1) * half
            ngrp = half // _L
            ts_ = [buft[ts][r, pl.ds(c0 + j * _L, _L)] for j in range(ngrp)]
            xs_ = [bufx[xs][r, pl.ds(c0 + j * _L, _L)] for j in range(ngrp)]
            for j in range(ngrp):
                bufx[xs][r, pl.ds(c0 + j * _L, _L)] = ts_[j] + xs_[j]

    t_copy(0, 0).start()
    x_copy(0, 0, 0).start()

    def chunk(i, carry):
        for gp in (0, 1):
            g = 2 * i + gp

            @pl.when(g + 1 < _N_CHUNKS)
            def _():
                t_copy(g + 1, 1 - gp).start()

            t_copy(g, gp).wait()

            for bb in range(_BATCH):
                xs = bb & 1
                ns = 1 - xs
                if bb == 0:

                    @pl.when(g >= 1)
                    def _():
                        out_copy(g - 1, 3, ns).wait()

                else:
                    out_copy(g, bb - 1, ns).wait()

                if bb < _BATCH - 1:
                    x_copy(g, bb + 1, ns).start()
                else:

                    @pl.when(g + 1 < _N_CHUNKS)
                    def _():
                        x_copy(g + 1, 0, ns).start()

                x_copy(g, bb, xs).wait()
                accumulate(xs, gp)
                out_copy(g, bb, xs).start()
        return carry

    lax.fori_loop(0, _N_CHUNKS // 2, chunk, 0)
    out_copy(_N_CHUNKS - 1, 3, 1).wait()


_sc_add = functools.partial(
    pl.kernel,
    out_type=jax.ShapeDtypeStruct((_BATCH, _SEQ, _D), jnp.float32),
    mesh=plsc.VectorSubcoreMesh(core_axis_name="c", subcore_axis_name="s"),
    compiler_params=pltpu.CompilerParams(use_tc_tiling_on_sc=True),
    scratch_types=[
        pltpu.VMEM((_R, _D), jnp.float32),
        pltpu.VMEM((_R, _D), jnp.float32),
        pltpu.VMEM((_R, _D), jnp.float32),
        pltpu.VMEM((_R, _D), jnp.float32),
        pltpu.SemaphoreType.DMA,
        pltpu.SemaphoreType.DMA,
        pltpu.SemaphoreType.DMA,
        pltpu.SemaphoreType.DMA,
        pltpu.SemaphoreType.DMA,
        pltpu.SemaphoreType.DMA,
    ],
)(_sc_body)


def kernel(x, pos_table):
    return _sc_add(x, pos_table)


# speedup vs baseline: 1.1425x; 1.1425x over previous
import functools

import jax
import jax.numpy as jnp
from jax import lax
from jax.experimental import pallas as pl
from jax.experimental.pallas import tpu as pltpu
from jax.experimental.pallas import tpu_sc as plsc

_BATCH = 4
_SEQ = 8192
_D = 768
_NW = 32
_ROWS_PER_W = _SEQ // _NW
_R = 16
_N_CHUNKS = _ROWS_PER_W // _R
_L = 16


def _sc_body(x_hbm, t_hbm, out_hbm,
             bx0, bx1, bx2, bx3, bt0, bt1,
             sx0, sx1, sx2, sx3, st0, st1, so0, so1, so2, so3):
    bufx = (bx0, bx1, bx2, bx3)
    buft = (bt0, bt1)
    sx = (sx0, sx1, sx2, sx3)
    st = (st0, st1)
    so = (so0, so1, so2, so3)

    wid = lax.axis_index("s") * 2 + lax.axis_index("c")
    r0 = wid * _ROWS_PER_W

    def x_copy(g, bb, s):
        row = r0 + g * _R
        return pltpu.make_async_copy(
            x_hbm.at[bb, pl.ds(row, _R), :], bufx[s], sx[s])

    def t_copy(g, s):
        row = r0 + g * _R
        return pltpu.make_async_copy(
            t_hbm.at[pl.ds(row, _R), :], buft[s], st[s])

    def out_copy(g, bb, s):
        row = r0 + g * _R
        return pltpu.make_async_copy(
            bufx[s], out_hbm.at[bb, pl.ds(row, _R), :], so[s])

    def accumulate(s, ts):
        half = _D // 2

        @plsc.parallel_loop(0, 2 * _R, 1, unroll=2)
        def _(r2):
            r = r2 >> 1
            c0 = (r2 & 1) * half
            vs = [buft[ts][r, pl.ds(c0 + j * _L, _L)]
                  for j in range(half // _L)]
            for j, v in enumerate(vs):
                plsc.addupdate(bufx[s].at[r, pl.ds(c0 + j * _L, _L)], v)

    t_copy(0, 0).start()
    x_copy(0, 0, 0).start()
    x_copy(0, 1, 1).start()

    def chunk(i, carry):
        for gp in (0, 1):
            g = 2 * i + gp

            @pl.when(g + 1 < _N_CHUNKS)
            def _():
                t_copy(g + 1, 1 - gp).start()

            t_copy(g, gp).wait()

            for bb in range(_BATCH):
                ps = (bb + 2) % 4
                if bb < 2:

                    @pl.when(g >= 1)
                    def _():
                        out_copy(g - 1, bb + 2, ps).wait()

                    x_copy(g, bb + 2, ps).start()
                else:
                    out_copy(g, bb - 2, ps).wait()

                    @pl.when(g + 1 < _N_CHUNKS)
                    def _():
                        x_copy(g + 1, bb - 2, ps).start()

                x_copy(g, bb, bb).wait()
                accumulate(bb, gp)
                out_copy(g, bb, bb).start()
        return carry

    lax.fori_loop(0, _N_CHUNKS // 2, chunk, 0)
    out_copy(_N_CHUNKS - 1, 2, 2).wait()
    out_copy(_N_CHUNKS - 1, 3, 3).wait()


_sc_add = functools.partial(
    pl.kernel,
    out_type=jax.ShapeDtypeStruct((_BATCH, _SEQ, _D), jnp.float32),
    mesh=plsc.VectorSubcoreMesh(core_axis_name="c", subcore_axis_name="s"),
    compiler_params=pltpu.CompilerParams(use_tc_tiling_on_sc=True),
    scratch_types=(
        [pltpu.VMEM((_R, _D), jnp.float32)] * 6
        + [pltpu.SemaphoreType.DMA] * 10
    ),
)(_sc_body)


def kernel(x, pos_table):
    return _sc_add(x, pos_table)
